# baseline (speedup 1.0000x reference)
import functools

import jax
import jax.numpy as jnp
from jax import lax
from jax.experimental import pallas as pl
from jax.experimental.pallas import tpu as pltpu
from jax.experimental.pallas import tpu_sc as plsc

EMBED = 49152
BATCH_ROWS = 2048
NC, NS = 2, 16
NW = NC * NS
SPW = BATCH_ROWS // NW


def _scatter_sorted(vals, order, nrun, preval, table):
    mesh = plsc.VectorSubcoreMesh(core_axis_name="c", subcore_axis_name="s")

    @functools.partial(
        pl.kernel,
        mesh=mesh,
        out_type=jax.ShapeDtypeStruct((BATCH_ROWS, EMBED), jnp.float32),
        scratch_types=[
            pltpu.VMEM((SPW + 16,), jnp.int32),
            pltpu.VMEM((SPW + 16,), jnp.int32),
            pltpu.VMEM((SPW + 16,), jnp.int32),
            pltpu.VMEM((SPW + 16,), jnp.int32),
            pltpu.VMEM((2, EMBED), jnp.float32),
            pltpu.SemaphoreType.DMA,
            pltpu.SemaphoreType.DMA,
            pltpu.SemaphoreType.DMA,
        ],
    )
    def k(vals_hbm, order_hbm, nrun_hbm, preval_hbm, table_hbm, out_hbm,
          vals_v, order_v, nrun_v, preval_v, buf,
          psem, wsem0, wsem1):
        wid = lax.axis_index("s") * NC + lax.axis_index("c")
        base = wid * SPW
        for hbm, vm in ((vals_hbm, vals_v), (order_hbm, order_v),
                        (nrun_hbm, nrun_v), (preval_hbm, preval_v)):
            pltpu.sync_copy(hbm.at[pl.ds(base, SPW)], vm.at[pl.ds(0, SPW)])

        wsems = (wsem0, wsem1)

        def ext(ref, j):
            return ref[pl.ds(j, 16)][0]

        def wwait(sem):
            def w(i, c):
                pltpu.make_async_copy(buf.at[pl.ds(0, 1)],
                                      out_hbm.at[pl.ds(0, 1)], sem).wait()
                return c
            return w

        def slot(j, carry):
            p, n0, n1 = carry
            r = ext(order_v, j)
            nr = ext(nrun_v, j)
            pv = ext(preval_v, j)

            pp = jnp.where(nr == 1, 1 - p, p)

            @pl.when((j == 0) & (nr == 1))
            def _():
                v0 = ext(vals_v, j)
                pltpu.sync_copy(table_hbm.at[pl.ds(v0, 1)],
                                buf.at[pl.ds(0, 1)])

            @pl.when((j > 0) & (nr == 1))
            def _():
                pltpu.make_async_copy(table_hbm.at[pl.ds(0, 1)],
                                      buf.at[pl.ds(0, 1)], psem).wait()

            ns = [n0, n1]
            for b in range(2):
                @pl.when(pp == b)
                def _(b=b):
                    pltpu.make_async_copy(
                        buf.at[pl.ds(b, 1)],
                        out_hbm.at[pl.ds(r, 1)], wsems[b]).start()
                ns[b] = ns[b] + jnp.where(pp == b, 1, 0).astype(jnp.int32)

            for b in range(2):
                @pl.when((pv >= 0) & (pp == 1 - b))
                def _(b=b):
                    lax.fori_loop(0, ns[b], wwait(wsems[b]), jnp.int32(0))
                    pltpu.make_async_copy(table_hbm.at[pl.ds(pv, 1)],
                                          buf.at[pl.ds(b, 1)], psem).start()
                ns[b] = jnp.where((pv >= 0) & (pp == 1 - b), 0,
                                  ns[b]).astype(jnp.int32)

            return (pp, ns[0], ns[1])

        _, n0, n1 = lax.fori_loop(
            0, SPW, slot,
            (jnp.int32(1), jnp.int32(0), jnp.int32(0)))
        lax.fori_loop(0, n0, wwait(wsem0), jnp.int32(0))
        lax.fori_loop(0, n1, wwait(wsem1), jnp.int32(0))

    return k(vals, order, nrun, preval, table)


def kernel(prefix, table):
    idx = prefix.reshape(-1).astype(jnp.int32)
    order = jnp.argsort(idx).astype(jnp.int32)
    vals = jnp.take(idx, order)

    jw = jnp.arange(BATCH_ROWS, dtype=jnp.int32)
    prev = jnp.concatenate([jnp.full((1,), -1, jnp.int32), vals[:-1]])
    nrun = ((jw % SPW == 0) | (vals != prev)).astype(jnp.int32)

    cand = jnp.where(nrun == 1, jw, jnp.int32(BATCH_ROWS))
    sufmin = lax.cummin(cand[::-1])[::-1]
    nb = jnp.concatenate([sufmin[1:], jnp.full((1,), BATCH_ROWS, jnp.int32)])
    valid = (nb < BATCH_ROWS) & (nb // SPW == jw // SPW)
    preval = jnp.where(
        (nrun == 1) & valid,
        jnp.take(vals, jnp.clip(nb, 0, BATCH_ROWS - 1)),
        jnp.int32(-1))

    out = _scatter_sorted(vals, order, nrun, preval, table)
    return out.reshape(prefix.shape[0], prefix.shape[1], EMBED)


# speedup vs baseline: 1.0453x; 1.0453x over previous
import functools

import jax
import jax.numpy as jnp
from jax import lax
from jax.experimental import pallas as pl
from jax.experimental.pallas import tpu as pltpu
from jax.experimental.pallas import tpu_sc as plsc

EMBED = 49152
BATCH_ROWS = 2048
NC, NS = 2, 16
NW = NC * NS
SPW = BATCH_ROWS // NW


def _scatter_sorted(idx, order, sched, table):
    mesh = plsc.VectorSubcoreMesh(core_axis_name="c", subcore_axis_name="s")

    @functools.partial(
        pl.kernel,
        mesh=mesh,
        out_type=jax.ShapeDtypeStruct((BATCH_ROWS, EMBED), jnp.float32),
        scratch_types=[
            pltpu.VMEM((BATCH_ROWS + 16,), jnp.int32),
            pltpu.VMEM((SPW + 16,), jnp.int32),
            pltpu.VMEM((SPW + 16,), jnp.int32),
            pltpu.VMEM((2, EMBED), jnp.float32),
            pltpu.SemaphoreType.DMA,
            pltpu.SemaphoreType.DMA,
            pltpu.SemaphoreType.DMA,
        ],
    )
    def k(idx_hbm, order_hbm, sched_hbm, table_hbm, out_hbm,
          idx_v, order_v, sched_v, buf, psem, wsem0, wsem1):
        wid = lax.axis_index("s") * NC + lax.axis_index("c")
        base = wid * SPW
        pltpu.sync_copy(idx_hbm, idx_v.at[pl.ds(0, BATCH_ROWS)])
        pltpu.sync_copy(order_hbm.at[pl.ds(base, SPW)],
                        order_v.at[pl.ds(0, SPW)])
        pltpu.sync_copy(sched_hbm.at[pl.ds(base, SPW)],
                        sched_v.at[pl.ds(0, SPW)])

        wsems = (wsem0, wsem1)

        def ext(ref, j):
            return ref[pl.ds(j, 16)][0]

        def wwait(sem):
            def w(i, c):
                pltpu.make_async_copy(buf.at[pl.ds(0, 1)],
                                      out_hbm.at[pl.ds(0, 1)], sem).wait()
                return c
            return w

        def slot(j, carry):
            p, n0, n1 = carry
            r = ext(order_v, j)
            sc = ext(sched_v, j)
            nr = sc >= -1

            pp = jnp.where(nr, 1 - p, p)

            @pl.when((j == 0) & nr)
            def _():
                v0 = ext(idx_v, r)
                pltpu.sync_copy(table_hbm.at[pl.ds(v0, 1)],
                                buf.at[pl.ds(0, 1)])

            @pl.when((j > 0) & nr)
            def _():
                pltpu.make_async_copy(table_hbm.at[pl.ds(0, 1)],
                                      buf.at[pl.ds(0, 1)], psem).wait()

            ns = [n0, n1]
            for b in range(2):
                @pl.when(pp == b)
                def _(b=b):
                    pltpu.make_async_copy(
                        buf.at[pl.ds(b, 1)],
                        out_hbm.at[pl.ds(r, 1)], wsems[b]).start()
                ns[b] = ns[b] + jnp.where(pp == b, 1, 0).astype(jnp.int32)

            loc = lax.max(sc - base, jnp.int32(0))
            rn = ext(order_v, loc)
            pv = ext(idx_v, lax.min(lax.max(rn, jnp.int32(0)),
                                    jnp.int32(BATCH_ROWS - 1)))
            for b in range(2):
                @pl.when((sc >= 0) & (pp == 1 - b))
                def _(b=b):
                    lax.fori_loop(0, ns[b], wwait(wsems[b]), jnp.int32(0))
                    pltpu.make_async_copy(table_hbm.at[pl.ds(pv, 1)],
                                          buf.at[pl.ds(b, 1)], psem).start()
                ns[b] = jnp.where((sc >= 0) & (pp == 1 - b), 0,
                                  ns[b]).astype(jnp.int32)

            return (pp, ns[0], ns[1])

        _, n0, n1 = lax.fori_loop(
            0, SPW, slot,
            (jnp.int32(1), jnp.int32(0), jnp.int32(0)))
        lax.fori_loop(0, n0, wwait(wsem0), jnp.int32(0))
        lax.fori_loop(0, n1, wwait(wsem1), jnp.int32(0))

    return k(idx, order, sched, table)


def kernel(prefix, table):
    idx = prefix.reshape(-1).astype(jnp.int32)
    vals, order = lax.sort((idx, jnp.arange(BATCH_ROWS, dtype=jnp.int32)),
                           num_keys=1)

    jw = jnp.arange(BATCH_ROWS, dtype=jnp.int32)
    prev = jnp.concatenate([jnp.full((1,), -1, jnp.int32), vals[:-1]])
    nrun = (jw % SPW == 0) | (vals != prev)

    cand = jnp.where(nrun, jw, jnp.int32(BATCH_ROWS))
    sufmin = lax.cummin(cand[::-1])[::-1]
    nb = jnp.concatenate([sufmin[1:], jnp.full((1,), BATCH_ROWS, jnp.int32)])
    valid = (nb < BATCH_ROWS) & (nb // SPW == jw // SPW)
    sched = jnp.where(~nrun, jnp.int32(-2),
                      jnp.where(valid, nb, jnp.int32(-1)))

    out = _scatter_sorted(idx, order.astype(jnp.int32), sched, table)
    return out.reshape(prefix.shape[0], prefix.shape[1], EMBED)
